# baseline (speedup 1.0000x reference)
import functools

import jax
import jax.numpy as jnp
from jax import lax
from jax.experimental import pallas as pl
from jax.experimental.pallas import tpu as pltpu
from jax.experimental.pallas import tpu_sc as plsc

_VOCAB = 8192
_B, _T = 16, 512
_N = _B * _T
_NC, _NS = 2, 16
_NW = _NC * _NS
_RPW = _N // _NW
_CH = 2
_NBUF = 4
_LAG = 2
_NCH = _RPW // _CH

_mesh = plsc.VectorSubcoreMesh(core_axis_name="c", subcore_axis_name="s")


@functools.partial(
    pl.kernel,
    mesh=_mesh,
    out_type=jax.ShapeDtypeStruct((_N, _VOCAB), jnp.float32),
    scratch_types=[
        pltpu.VMEM((_NCH, _CH), jnp.int32),
        pltpu.VMEM((_CH, _VOCAB), jnp.float32),
        pltpu.VMEM((_CH, _VOCAB), jnp.float32),
        pltpu.VMEM((_CH, _VOCAB), jnp.float32),
        pltpu.VMEM((_CH, _VOCAB), jnp.float32),
        pltpu.SemaphoreType.DMA,
        pltpu.SemaphoreType.DMA,
        pltpu.SemaphoreType.DMA,
        pltpu.SemaphoreType.DMA,
        pltpu.SemaphoreType.DMA,
        pltpu.SemaphoreType.DMA,
        pltpu.SemaphoreType.DMA,
        pltpu.SemaphoreType.DMA,
    ],
)
def _gather_rows(x_hbm, table_hbm, out_hbm, idx_v,
                 buf0, buf1, buf2, buf3, g0, g1, g2, g3, s0, s1, s2, s3):
    wid = lax.axis_index("s") * _NC + lax.axis_index("c")
    base = wid * _RPW
    bufs = (buf0, buf1, buf2, buf3)
    gsems = (g0, g1, g2, g3)
    ssems = (s0, s1, s2, s3)

    pltpu.sync_copy(x_hbm.at[wid], idx_v)

    def gdesc(chunk, b):
        return pltpu.make_async_copy(
            table_hbm.at[idx_v.at[chunk]], bufs[b], gsems[b])

    def sdesc(chunk, b):
        return pltpu.make_async_copy(
            bufs[b], out_hbm.at[pl.ds(base + chunk * _CH, _CH)], ssems[b])

    for b in range(_NBUF):
        gdesc(b, b).start()

    for j in range(_LAG):
        gdesc(j, j % _NBUF).wait()
        sdesc(j, j % _NBUF).start()

    def body(o, carry):
        for i in range(_NBUF):
            j = o * _NBUF + i + _LAG
            b = (i + _LAG) % _NBUF
            gdesc(j, b).wait()
            sdesc(j, b).start()
            t = j - _LAG
            bt = i % _NBUF
            sdesc(t, bt).wait()
            gdesc(t + _NBUF, bt).start()
        return carry

    lax.fori_loop(0, (_NCH - _NBUF) // _NBUF, body, 0)

    for j in range(_NCH - _NBUF + _LAG, _NCH):
        gdesc(j, j % _NBUF).wait()
        sdesc(j, j % _NBUF).start()

    for t in range(_NCH - _NBUF, _NCH):
        sdesc(t, t % _NBUF).wait()


def kernel(X, table):
    xf = X.reshape(_NW, _NCH, _CH).astype(jnp.int32)
    out = _gather_rows(xf, table)
    return out.reshape(_B, _T, _VOCAB)


# speedup vs baseline: 1.0111x; 1.0111x over previous
import functools

import jax
import jax.numpy as jnp
from jax import lax
from jax.experimental import pallas as pl
from jax.experimental.pallas import tpu as pltpu
from jax.experimental.pallas import tpu_sc as plsc

_VOCAB = 8192
_B, _T = 16, 512
_N = _B * _T
_NC, _NS = 2, 16
_NW = _NC * _NS
_RPW = _N // _NW
_CH = 4
_NBUF = 3
_LAG = 1
_NCH = _RPW // _CH
_MAIN = (_NCH - _NBUF) // _NBUF

_mesh = plsc.VectorSubcoreMesh(core_axis_name="c", subcore_axis_name="s")


@functools.partial(
    pl.kernel,
    mesh=_mesh,
    out_type=jax.ShapeDtypeStruct((_N, _VOCAB), jnp.float32),
    scratch_types=(
        [pltpu.VMEM((_NCH, _CH), jnp.int32)]
        + [pltpu.VMEM((_CH, _VOCAB), jnp.float32)] * _NBUF
        + [pltpu.SemaphoreType.DMA] * (2 * _NBUF)
    ),
)
def _gather_rows(x_hbm, table_hbm, out_hbm, idx_v, *bufs_and_sems):
    bufs = bufs_and_sems[:_NBUF]
    gsems = bufs_and_sems[_NBUF:2 * _NBUF]
    ssems = bufs_and_sems[2 * _NBUF:]
    wid = lax.axis_index("s") * _NC + lax.axis_index("c")
    base = wid * _RPW

    pltpu.sync_copy(x_hbm.at[wid], idx_v)

    def gdesc(chunk, b):
        return pltpu.make_async_copy(
            table_hbm.at[idx_v.at[chunk]], bufs[b], gsems[b])

    def sdesc(chunk, b):
        return pltpu.make_async_copy(
            bufs[b], out_hbm.at[pl.ds(base + chunk * _CH, _CH)], ssems[b])

    for b in range(_NBUF):
        gdesc(b, b).start()

    for j in range(_LAG):
        gdesc(j, j % _NBUF).wait()
        sdesc(j, j % _NBUF).start()

    def body(o, carry):
        for i in range(_NBUF):
            j = o * _NBUF + i + _LAG
            b = (i + _LAG) % _NBUF
            gdesc(j, b).wait()
            sdesc(j, b).start()
            t = j - _LAG
            bt = i % _NBUF
            sdesc(t, bt).wait()
            gdesc(t + _NBUF, bt).start()
        return carry

    lax.fori_loop(0, _MAIN, body, 0)

    for j in range(_LAG + _MAIN * _NBUF, _NCH):
        gdesc(j, j % _NBUF).wait()
        sdesc(j, j % _NBUF).start()
        t = j - _LAG
        if t + _NBUF < _NCH:
            sdesc(t, t % _NBUF).wait()
            gdesc(t + _NBUF, t % _NBUF).start()

    for t in range(_NCH - _NBUF, _NCH):
        sdesc(t, t % _NBUF).wait()


def kernel(X, table):
    xf = X.reshape(_NW, _NCH, _CH).astype(jnp.int32)
    out = _gather_rows(xf, table)
    return out.reshape(_B, _T, _VOCAB)
